# baseline (speedup 1.0000x reference)
import functools

import jax
import jax.numpy as jnp
from jax import lax
from jax.experimental import pallas as pl
from jax.experimental.pallas import tpu as pltpu
from jax.experimental.pallas import tpu_sc as plsc

VOCAB = 1000000
DIM = 64
BATCH = 4096
HIST = 200
N = BATCH * HIST
NC, NS = 2, 16
NW = NC * NS
SLAB = 128
NSLAB = VOCAB // SLAB
TAILV = VOCAB - NSLAB * SLAB
MD_ROWS = VOCAB // 2
C = 128
NBL = 3
NBO = 2


def _make_merge_pack():
    mesh = plsc.VectorSubcoreMesh(core_axis_name="c", subcore_axis_name="s")

    @functools.partial(
        pl.kernel,
        out_type=jax.ShapeDtypeStruct((MD_ROWS, 2 * DIM), jnp.float32),
        mesh=mesh,
        scratch_types=[pltpu.VMEM((DIM, SLAB), jnp.float32)] * 2
        + [pltpu.VMEM((DIM, SLAB), jnp.float32)] * 2
        + [pltpu.VMEM((SLAB // 2, 2 * DIM), jnp.float32)] * 2
        + [
            pltpu.VMEM((DIM, TAILV), jnp.float32),
            pltpu.VMEM((DIM, TAILV), jnp.float32),
            pltpu.VMEM((TAILV // 2, 2 * DIM), jnp.float32),
            pltpu.VMEM((2, 16), jnp.float32),
        ]
        + [pltpu.SemaphoreType.DMA] * 5,
        compiler_params=pltpu.CompilerParams(needs_layout_passes=False),
    )
    def merge_pack(bt_hbm, mt_hbm, coef_hbm, md_hbm, *scratch):
        bb = scratch[0:2]
        mb = scratch[2:4]
        ob = scratch[4:6]
        btail, mtail, otail, cvec = scratch[6:10]
        isem = scratch[10:12]
        osem = scratch[12:14]
        tsem = scratch[14]
        wid = lax.axis_index("s") * NC + lax.axis_index("c")
        pltpu.sync_copy(coef_hbm, cvec)
        c0 = cvec[0, :]
        c1 = cvec[1, :]
        iot = lax.iota(jnp.int32, 16)

        nt = (NSLAB - wid + NW - 1) // NW

        def start_in(t, p):
            v0 = pl.multiple_of((wid + NW * t) * SLAB, SLAB)
            pltpu.async_copy(bt_hbm.at[:, pl.ds(v0, SLAB)], bb[p], isem[p])
            pltpu.async_copy(mt_hbm.at[:, pl.ds(v0, SLAB)], mb[p], isem[p])

        def wait_in(t, p):
            v0 = pl.multiple_of((wid + NW * t) * SLAB, SLAB)
            pltpu.make_async_copy(bt_hbm.at[:, pl.ds(v0, SLAB)], bb[p], isem[p]).wait()
            pltpu.make_async_copy(mt_hbm.at[:, pl.ds(v0, SLAB)], mb[p], isem[p]).wait()

        def md_slice(t):
            k0 = pl.multiple_of((wid + NW * t) * (SLAB // 2), SLAB // 2)
            return md_hbm.at[pl.ds(k0, SLAB // 2)]

        def merge_rows(nrows, src_b, src_m, dst):
            def row(k, rc):
                for half in range(2):
                    jv = jnp.full((16,), 2 * k + half, jnp.int32)
                    for d0 in range(0, DIM, 16):
                        dv = iot + d0
                        v = (c0 * plsc.load_gather(src_b, [dv, jv])
                             + c1 * plsc.load_gather(src_m, [dv, jv]))
                        dst[k, pl.ds(half * DIM + d0, 16)] = v
                return rc

            lax.fori_loop(0, nrows, row, 0, unroll=2)

        start_in(0, 0)

        def body(t, carry):
            tp = lax.rem(t, 2)

            def for_p(p):
                wait_in(t, p)

                @pl.when(t + 1 < nt)
                def _():
                    start_in(t + 1, 1 - p)

                @pl.when(t >= 2)
                def _():
                    pltpu.make_async_copy(ob[p], md_slice(t - 2), osem[p]).wait()

                merge_rows(SLAB // 2, bb[p], mb[p], ob[p])
                pltpu.async_copy(ob[p], md_slice(t), osem[p])

            @pl.when(tp == 0)
            def _():
                for_p(0)

            @pl.when(tp == 1)
            def _():
                for_p(1)

            return carry

        lax.fori_loop(0, nt, body, 0)

        for back in (2, 1):
            t_last = nt - back

            @pl.when(t_last >= 0)
            def _(t_last=t_last):
                for p in range(2):
                    @pl.when(lax.rem(t_last, 2) == p)
                    def _(t_last=t_last, p=p):
                        pltpu.make_async_copy(ob[p], md_slice(t_last), osem[p]).wait()

        @pl.when(wid == NW - 1)
        def _():
            v0 = NSLAB * SLAB
            pltpu.sync_copy(bt_hbm.at[:, pl.ds(v0, TAILV)], btail)
            pltpu.sync_copy(mt_hbm.at[:, pl.ds(v0, TAILV)], mtail)
            merge_rows(TAILV // 2, btail, mtail, otail)
            pltpu.async_copy(
                otail, md_hbm.at[pl.ds(NSLAB * (SLAB // 2), TAILV // 2)], tsem
            ).wait()

    return merge_pack


def _make_gather_t():
    mesh = plsc.VectorSubcoreMesh(core_axis_name="c", subcore_axis_name="s")

    @functools.partial(
        pl.kernel,
        out_type=jax.ShapeDtypeStruct((HIST, DIM, BATCH), jnp.float32),
        mesh=mesh,
        scratch_types=[
            pltpu.VMEM((HIST, C), jnp.int32),
        ]
        + [pltpu.VMEM((C, 2 * DIM), jnp.float32)] * NBL
        + [pltpu.VMEM((C,), jnp.int32)] * NBL
        + [pltpu.VMEM((C,), jnp.int32)] * NBL
        + [pltpu.VMEM((DIM, C), jnp.float32)] * NBO
        + [pltpu.SemaphoreType.DMA] * (NBL + NBO),
        compiler_params=pltpu.CompilerParams(needs_layout_passes=False),
    )
    def gather_t(idxt_hbm, md_hbm, out_hbm, ibig, *scratch):
        lines = scratch[0:NBL]
        lbuf = scratch[NBL:2 * NBL]
        pbuf = scratch[2 * NBL:3 * NBL]
        tb = scratch[3 * NBL:3 * NBL + NBO]
        gsem = scratch[3 * NBL + NBO:3 * NBL + NBO + NBL]
        osem = scratch[3 * NBL + NBO + NBL:]
        wid = lax.axis_index("s") * NC + lax.axis_index("c")
        b0 = pl.multiple_of(wid * C, C)
        pltpu.sync_copy(idxt_hbm.at[:, pl.ds(b0, C)], ibig)
        iot = lax.iota(jnp.int32, 16)

        def prep_and_fire(h, s):
            def r0loop(r0, rc):
                iv = ibig[h, pl.ds(r0 * 16, 16)]
                lbuf[s][pl.ds(r0 * 16, 16)] = lax.shift_right_logical(iv, 1)
                pbuf[s][pl.ds(r0 * 16, 16)] = lax.shift_left(
                    lax.bitwise_and(iv, 1), 6)
                return rc

            lax.fori_loop(0, C // 16, r0loop, 0, unroll=8)
            pltpu.async_copy(md_hbm.at[lbuf[s]], lines[s], gsem[s])

        def wait_gather(s):
            pltpu.make_async_copy(md_hbm.at[lbuf[s]], lines[s], gsem[s]).wait()

        def out_slice(h):
            return out_hbm.at[h, :, pl.ds(b0, C)]

        for s in range(NBL):
            prep_and_fire(s, s)

        def body(h, carry):
            hs = lax.rem(h, NBL)
            hq = lax.rem(h, NBO)

            def do(s, q):
                wait_gather(s)

                @pl.when(h >= NBO)
                def _():
                    pltpu.make_async_copy(tb[q], out_slice(h - NBO), osem[q]).wait()

                for r0 in range(C // 16):
                    rv = iot + r0 * 16
                    pcv = pbuf[s][pl.ds(r0 * 16, 16)]

                    def dloop(d, rc):
                        tb[q][d, pl.ds(r0 * 16, 16)] = plsc.load_gather(
                            lines[s], [rv, pcv + d])
                        return rc

                    lax.fori_loop(0, DIM, dloop, 0, unroll=8)

                pltpu.async_copy(tb[q], out_slice(h), osem[q])

                @pl.when(h + NBL < HIST)
                def _():
                    prep_and_fire(h + NBL, s)

            for si in range(NBL):
                for qi in range(NBO):
                    @pl.when(jnp.logical_and(hs == si, hq == qi))
                    def _(si=si, qi=qi):
                        do(si, qi)

            return carry

        lax.fori_loop(0, HIST, body, 0)

        for hh in (HIST - 2, HIST - 1):
            q = hh % NBO
            pltpu.make_async_copy(tb[q], out_slice(hh), osem[q]).wait()

    return gather_t


_merge_pack = _make_merge_pack()
_gather_t = _make_gather_t()


def kernel(input, base_weight, mod_weight_0, mod_weight_1, merging_coefficients):
    del mod_weight_1
    idxt = input.T.astype(jnp.int32)
    bt = base_weight.T
    mt = mod_weight_0.T
    coefs = jnp.broadcast_to(
        merging_coefficients.astype(jnp.float32)[:, None], (2, 16)
    )
    md = _merge_pack(bt, mt, coefs)
    out_t = _gather_t(idxt, md)
    return jnp.transpose(out_t, (2, 0, 1))


# speedup vs baseline: 1.0611x; 1.0611x over previous
import functools

import jax
import jax.numpy as jnp
from jax import lax
from jax.experimental import pallas as pl
from jax.experimental.pallas import tpu as pltpu
from jax.experimental.pallas import tpu_sc as plsc

VOCAB = 1000000
DIM = 64
BATCH = 4096
HIST = 200
N = BATCH * HIST
NC, NS = 2, 16
NW = NC * NS
SLAB = 128
NSLAB = VOCAB // SLAB
TAILV = VOCAB - NSLAB * SLAB
MD_ROWS = VOCAB // 2
C = 128
NBL = 3
NBO = 2


def _make_merge_pack():
    mesh = plsc.VectorSubcoreMesh(core_axis_name="c", subcore_axis_name="s")

    @functools.partial(
        pl.kernel,
        out_type=jax.ShapeDtypeStruct((MD_ROWS, 2 * DIM), jnp.float32),
        mesh=mesh,
        scratch_types=[pltpu.VMEM((DIM, SLAB + 1), jnp.float32)] * 2
        + [pltpu.VMEM((DIM, SLAB + 1), jnp.float32)] * 2
        + [pltpu.VMEM((SLAB // 2, 2 * DIM), jnp.float32)] * 2
        + [
            pltpu.VMEM((DIM, SLAB + 1), jnp.float32),
            pltpu.VMEM((DIM, SLAB + 1), jnp.float32),
            pltpu.VMEM((TAILV // 2, 2 * DIM), jnp.float32),
            pltpu.VMEM((2, 16), jnp.float32),
        ]
        + [pltpu.SemaphoreType.DMA] * 5,
        compiler_params=pltpu.CompilerParams(needs_layout_passes=False),
    )
    def merge_pack(bt_hbm, mt_hbm, bttail_hbm, mttail_hbm, coef_hbm, md_hbm, *scratch):
        bb = scratch[0:2]
        mb = scratch[2:4]
        ob = scratch[4:6]
        btail, mtail, otail, cvec = scratch[6:10]
        isem = scratch[10:12]
        osem = scratch[12:14]
        tsem = scratch[14]
        wid = lax.axis_index("s") * NC + lax.axis_index("c")
        pltpu.sync_copy(coef_hbm, cvec)
        c0 = cvec[0, :]
        c1 = cvec[1, :]
        iot = lax.iota(jnp.int32, 16)

        nt = (NSLAB - wid + NW - 1) // NW

        def start_in(t, p):
            v0 = pl.multiple_of((wid + NW * t) * SLAB, SLAB)
            pltpu.async_copy(bt_hbm.at[:, pl.ds(v0, SLAB)], bb[p].at[:, pl.ds(0, SLAB)], isem[p])
            pltpu.async_copy(mt_hbm.at[:, pl.ds(v0, SLAB)], mb[p].at[:, pl.ds(0, SLAB)], isem[p])

        def wait_in(t, p):
            v0 = pl.multiple_of((wid + NW * t) * SLAB, SLAB)
            pltpu.make_async_copy(bt_hbm.at[:, pl.ds(v0, SLAB)], bb[p].at[:, pl.ds(0, SLAB)], isem[p]).wait()
            pltpu.make_async_copy(mt_hbm.at[:, pl.ds(v0, SLAB)], mb[p].at[:, pl.ds(0, SLAB)], isem[p]).wait()

        def md_slice(t):
            k0 = pl.multiple_of((wid + NW * t) * (SLAB // 2), SLAB // 2)
            return md_hbm.at[pl.ds(k0, SLAB // 2)]

        def merge_rows(nrows, src_b, src_m, dst):
            def row(k, rc):
                for half in range(2):
                    jv = jnp.full((16,), 2 * k + half, jnp.int32)
                    for d0 in range(0, DIM, 16):
                        dv = iot + d0
                        v = (c0 * plsc.load_gather(src_b, [dv, jv])
                             + c1 * plsc.load_gather(src_m, [dv, jv]))
                        dst[k, pl.ds(half * DIM + d0, 16)] = v
                return rc

            lax.fori_loop(0, nrows, row, 0, unroll=2)

        start_in(0, 0)

        def body(t, carry):
            tp = lax.rem(t, 2)

            def for_p(p):
                wait_in(t, p)

                @pl.when(t + 1 < nt)
                def _():
                    start_in(t + 1, 1 - p)

                @pl.when(t >= 2)
                def _():
                    pltpu.make_async_copy(ob[p], md_slice(t - 2), osem[p]).wait()

                merge_rows(SLAB // 2, bb[p], mb[p], ob[p])
                pltpu.async_copy(ob[p], md_slice(t), osem[p])

            @pl.when(tp == 0)
            def _():
                for_p(0)

            @pl.when(tp == 1)
            def _():
                for_p(1)

            return carry

        lax.fori_loop(0, nt, body, 0)

        for back in (2, 1):
            t_last = nt - back

            @pl.when(t_last >= 0)
            def _(t_last=t_last):
                for p in range(2):
                    @pl.when(lax.rem(t_last, 2) == p)
                    def _(t_last=t_last, p=p):
                        pltpu.make_async_copy(ob[p], md_slice(t_last), osem[p]).wait()

        @pl.when(wid == NW - 1)
        def _():
            pltpu.sync_copy(bttail_hbm, btail.at[:, pl.ds(0, SLAB)])
            pltpu.sync_copy(mttail_hbm, mtail.at[:, pl.ds(0, SLAB)])
            merge_rows(TAILV // 2, btail, mtail, otail)
            pltpu.async_copy(
                otail, md_hbm.at[pl.ds(NSLAB * (SLAB // 2), TAILV // 2)], tsem
            ).wait()

    return merge_pack


def _make_gather_t():
    mesh = plsc.VectorSubcoreMesh(core_axis_name="c", subcore_axis_name="s")

    @functools.partial(
        pl.kernel,
        out_type=jax.ShapeDtypeStruct((HIST, DIM, BATCH), jnp.float32),
        mesh=mesh,
        scratch_types=[
            pltpu.VMEM((HIST, C), jnp.int32),
        ]
        + [pltpu.VMEM((C, 2 * DIM), jnp.float32)] * NBL
        + [pltpu.VMEM((C,), jnp.int32)] * NBL
        + [pltpu.VMEM((C,), jnp.int32)] * NBL
        + [pltpu.VMEM((DIM, C + 1), jnp.float32)] * NBO
        + [pltpu.SemaphoreType.DMA] * (NBL + NBO),
        compiler_params=pltpu.CompilerParams(needs_layout_passes=False),
    )
    def gather_t(idxt_hbm, md_hbm, out_hbm, ibig, *scratch):
        lines = scratch[0:NBL]
        lbuf = scratch[NBL:2 * NBL]
        pbuf = scratch[2 * NBL:3 * NBL]
        tb = scratch[3 * NBL:3 * NBL + NBO]
        gsem = scratch[3 * NBL + NBO:3 * NBL + NBO + NBL]
        osem = scratch[3 * NBL + NBO + NBL:]
        wid = lax.axis_index("s") * NC + lax.axis_index("c")
        b0 = pl.multiple_of(wid * C, C)
        pltpu.sync_copy(idxt_hbm.at[:, pl.ds(b0, C)], ibig)
        iot = lax.iota(jnp.int32, 16)

        def prep_and_fire(h, s):
            def r0loop(r0, rc):
                iv = ibig[h, pl.ds(r0 * 16, 16)]
                lbuf[s][pl.ds(r0 * 16, 16)] = lax.shift_right_logical(iv, 1)
                pbuf[s][pl.ds(r0 * 16, 16)] = lax.shift_left(
                    lax.bitwise_and(iv, 1), 6)
                return rc

            lax.fori_loop(0, C // 16, r0loop, 0, unroll=8)
            pltpu.async_copy(md_hbm.at[lbuf[s]], lines[s], gsem[s])

        def wait_gather(s):
            pltpu.make_async_copy(md_hbm.at[lbuf[s]], lines[s], gsem[s]).wait()

        def out_slice(h):
            return out_hbm.at[h, :, pl.ds(b0, C)]

        def tb_view(q):
            return tb[q].at[:, pl.ds(0, C)]

        for s in range(NBL):
            prep_and_fire(s, s)

        def body(h, carry):
            hs = lax.rem(h, NBL)
            hq = lax.rem(h, NBO)

            def do(s, q):
                wait_gather(s)

                @pl.when(h >= NBO)
                def _():
                    pltpu.make_async_copy(tb_view(q), out_slice(h - NBO), osem[q]).wait()

                def rblk(r0, rc):
                    parv = pbuf[s][pl.ds(pl.multiple_of(r0 * 16, 16), 16)]
                    for i in range(16):
                        r = r0 * 16 + i
                        po = parv[i]
                        rvec = jnp.full((16,), r, jnp.int32)
                        for d0 in range(0, DIM, 16):
                            v = lines[s][r, pl.ds(po + d0, 16)]
                            plsc.store_scatter(tb[q], [iot + d0, rvec], v)
                    return rc

                lax.fori_loop(0, C // 16, rblk, 0)

                pltpu.async_copy(tb_view(q), out_slice(h), osem[q])

                @pl.when(h + NBL < HIST)
                def _():
                    prep_and_fire(h + NBL, s)

            for si in range(NBL):
                for qi in range(NBO):
                    @pl.when(jnp.logical_and(hs == si, hq == qi))
                    def _(si=si, qi=qi):
                        do(si, qi)

            return carry

        lax.fori_loop(0, HIST, body, 0)

        for hh in (HIST - 2, HIST - 1):
            q = hh % NBO
            pltpu.make_async_copy(tb_view(q), out_slice(hh), osem[q]).wait()

    return gather_t


_merge_pack = _make_merge_pack()
_gather_t = _make_gather_t()


def kernel(input, base_weight, mod_weight_0, mod_weight_1, merging_coefficients):
    del mod_weight_1
    idxt = input.T.astype(jnp.int32)
    bt = base_weight.T
    mt = mod_weight_0.T
    coefs = jnp.broadcast_to(
        merging_coefficients.astype(jnp.float32)[:, None], (2, 16)
    )
    ntail = (VOCAB // SLAB) * SLAB
    bt_tail = jnp.pad(bt[:, ntail:], ((0, 0), (0, SLAB - TAILV)))
    mt_tail = jnp.pad(mt[:, ntail:], ((0, 0), (0, SLAB - TAILV)))
    md = _merge_pack(bt, mt, bt_tail, mt_tail, coefs)
    out_t = _gather_t(idxt, md)
    return jnp.transpose(out_t, (2, 0, 1))


# speedup vs baseline: 2.2965x; 2.1643x over previous
import functools

import jax
import jax.numpy as jnp
from jax import lax
from jax.experimental import pallas as pl
from jax.experimental.pallas import tpu as pltpu
from jax.experimental.pallas import tpu_sc as plsc

VOCAB = 1000000
DIM = 64
N = 4096 * 200
NC, NS = 2, 16
NW = NC * NS
BPW = N // NW
C = 128
NCHUNK = BPW // C
NB = 4


def _make_merged_gather():
    mesh = plsc.VectorSubcoreMesh(core_axis_name="c", subcore_axis_name="s")

    @functools.partial(
        pl.kernel,
        out_type=jax.ShapeDtypeStruct((N // 2, 2 * DIM), jnp.float32),
        mesh=mesh,
        scratch_types=[
            pltpu.VMEM((BPW,), jnp.int32),
        ]
        + [pltpu.VMEM((C, 2 * DIM), jnp.float32)] * NB
        + [
            pltpu.VMEM((2, 16), jnp.float32),
        ]
        + [pltpu.SemaphoreType.DMA] * (2 * NB),
    )
    def merged_gather(idx_hbm, pair_hbm, coef_hbm, out_hbm, idx_v, *scratch):
        lrows = scratch[:NB]
        cvec = scratch[NB]
        gsem = scratch[NB + 1:NB + 1 + NB]
        osem = scratch[NB + 1 + NB:]
        wid = lax.axis_index("s") * NC + lax.axis_index("c")
        row0 = wid * BPW
        pltpu.sync_copy(coef_hbm, cvec)
        pltpu.sync_copy(idx_hbm.at[pl.ds(row0, BPW)], idx_v)
        c0 = cvec[0, :]
        c1 = cvec[1, :]

        def start_gather(g, b):
            idx_slice = idx_v.at[pl.ds(g * C, C)]
            pltpu.async_copy(pair_hbm.at[idx_slice], lrows[b], gsem[b])

        def wait_gather(g, b):
            idx_slice = idx_v.at[pl.ds(g * C, C)]
            pltpu.make_async_copy(pair_hbm.at[idx_slice], lrows[b], gsem[b]).wait()

        def out_slice(g):
            off = pl.multiple_of((row0 + g * C) // 2, C // 2)
            return out_hbm.at[pl.ds(off, C // 2)]

        for b in range(NB):
            start_gather(b, b)

        def outer(w, carry):
            for b in range(NB):
                g = w * NB + b
                wait_gather(g, b)

                def row_body(r, rc):
                    for j in range(DIM // 16):
                        s = pl.ds(j * 16, 16)
                        sm = pl.ds(DIM + j * 16, 16)
                        m0 = c0 * lrows[b][2 * r, s] + c1 * lrows[b][2 * r, sm]
                        m1 = c0 * lrows[b][2 * r + 1, s] + c1 * lrows[b][2 * r + 1, sm]
                        lrows[b][r, s] = m0
                        lrows[b][r, sm] = m1
                    return rc

                lax.fori_loop(0, C // 2, row_body, 0, unroll=4)

                merged = lrows[b].at[pl.ds(0, C // 2)]
                pltpu.async_copy(merged, out_slice(g), osem[b])

                @pl.when(g + NB < NCHUNK)
                def _():
                    pltpu.make_async_copy(merged, out_slice(g), osem[b]).wait()
                    start_gather(g + NB, b)
            return carry

        lax.fori_loop(0, NCHUNK // NB, outer, 0)

        for b in range(NB):
            g = NCHUNK - NB + b
            merged = lrows[b].at[pl.ds(0, C // 2)]
            pltpu.make_async_copy(merged, out_slice(g), osem[b]).wait()

    return merged_gather


_merged_gather = _make_merged_gather()


def kernel(input, base_weight, mod_weight_0, mod_weight_1, merging_coefficients):
    del mod_weight_1
    idx = input.reshape(-1).astype(jnp.int32)
    pair = jnp.concatenate([base_weight, mod_weight_0], axis=1)
    coefs = jnp.broadcast_to(
        merging_coefficients.astype(jnp.float32)[:, None], (2, 16)
    )
    out = _merged_gather(idx, pair, coefs)
    return out.reshape(input.shape + (DIM,))
